# baseline (speedup 1.0000x reference)
import jax
import jax.numpy as jnp
from jax import lax
from jax.experimental import pallas as pl
from jax.experimental.pallas import tpu as pltpu
from jax.experimental.pallas import tpu_sc as plsc

_NC, _NS = 2, 16
_NW = _NC * _NS
_L = 16


def _gather_body(i_idx_hbm, j_idx_hbm, wi2_hbm, wj2_hbm, bi128_hbm, bj128_hbm,
                 out_i, out_j, out_bias,
                 idx_i_v, idx_j_v, pair_i_v, pair_j_v, brow_i_v, brow_j_v,
                 rows_i_v, rows_j_v, browi_v, browj_v, bias_v, sem):
    bpw = idx_i_v.shape[0]
    wid = lax.axis_index("s") * _NC + lax.axis_index("c")
    base = wid * bpw
    pltpu.sync_copy(i_idx_hbm.at[pl.ds(base, bpw)], idx_i_v)
    pltpu.sync_copy(j_idx_hbm.at[pl.ds(base, bpw)], idx_j_v)
    for k in range(bpw // _L):
        sl = pl.ds(k * _L, _L)
        ii = idx_i_v[sl]
        jj = idx_j_v[sl]
        pair_i_v[sl] = lax.shift_right_logical(ii, 1)
        pair_j_v[sl] = lax.shift_right_logical(jj, 1)
        brow_i_v[sl] = lax.shift_right_logical(ii, 7)
        brow_j_v[sl] = lax.shift_right_logical(jj, 7)
    c1 = pltpu.async_copy(wi2_hbm.at[pair_i_v], rows_i_v, sem)
    c2 = pltpu.async_copy(wj2_hbm.at[pair_j_v], rows_j_v, sem)
    c3 = pltpu.async_copy(bi128_hbm.at[brow_i_v], browi_v, sem)
    c4 = pltpu.async_copy(bj128_hbm.at[brow_j_v], browj_v, sem)
    c1.wait()
    c2.wait()
    c3.wait()
    c4.wait()
    for k in range(bpw // _L):
        sl = pl.ds(k * _L, _L)
        rid = lax.iota(jnp.int32, _L) + k * _L
        bi_vals = plsc.load_gather(browi_v, [rid, idx_i_v[sl] & 127])
        bj_vals = plsc.load_gather(browj_v, [rid, idx_j_v[sl] & 127])
        bias_v[sl] = bi_vals + bj_vals
    pltpu.sync_copy(rows_i_v, out_i.at[pl.ds(base, bpw)])
    pltpu.sync_copy(rows_j_v, out_j.at[pl.ds(base, bpw)])
    pltpu.sync_copy(bias_v, out_bias.at[pl.ds(base, bpw)])


def _sc_gather(i_idx, j_idx, wi2, wj2, bi128, bj128):
    b = i_idx.shape[0]
    w = wi2.shape[1]
    bpw = b // _NW
    mesh = plsc.VectorSubcoreMesh(core_axis_name="c", subcore_axis_name="s")
    return pl.kernel(
        _gather_body,
        out_type=(
            jax.ShapeDtypeStruct((b, w), jnp.float32),
            jax.ShapeDtypeStruct((b, w), jnp.float32),
            jax.ShapeDtypeStruct((b,), jnp.float32),
        ),
        mesh=mesh,
        scratch_types=(
            pltpu.VMEM((bpw,), jnp.int32),
            pltpu.VMEM((bpw,), jnp.int32),
            pltpu.VMEM((bpw,), jnp.int32),
            pltpu.VMEM((bpw,), jnp.int32),
            pltpu.VMEM((bpw,), jnp.int32),
            pltpu.VMEM((bpw,), jnp.int32),
            pltpu.VMEM((bpw, w), jnp.float32),
            pltpu.VMEM((bpw, w), jnp.float32),
            pltpu.VMEM((bpw, w), jnp.float32),
            pltpu.VMEM((bpw, w), jnp.float32),
            pltpu.VMEM((bpw,), jnp.float32),
            pltpu.SemaphoreType.DMA,
        ),
        compiler_params=pltpu.CompilerParams(needs_layout_passes=False),
    )(i_idx, j_idx, wi2, wj2, bi128, bj128)


def _matmul_body(a2_ref, b2_ref, mi_ref, mj_ref, bias_ref, o_ref, bsel_ref):
    d = mi_ref.shape[1]

    @pl.when(pl.program_id(0) == 0)
    def _():
        b2 = b2_ref[...]
        mj = mj_ref[...]
        bsel_ref[...] = b2[:, d:] * mj + b2[:, :d] * (1.0 - mj)

    a2 = a2_ref[...]
    mi = mi_ref[...]
    a = a2[:, d:] * mi + a2[:, :d] * (1.0 - mi)
    o_ref[...] = lax.dot_general(
        a, bsel_ref[...], (((1,), (1,)), ((), ())),
        preferred_element_type=jnp.float32,
        precision=lax.Precision.HIGHEST,
    ) + bias_ref[...]


def _tc_matmul(rows_i, rows_j, mask_i, mask_j, bias):
    b, w = rows_i.shape
    d = w // 2
    bm = 512
    return pl.pallas_call(
        _matmul_body,
        grid=(b // bm,),
        in_specs=[
            pl.BlockSpec((bm, w), lambda r: (r, 0)),
            pl.BlockSpec((b, w), lambda r: (0, 0)),
            pl.BlockSpec((bm, d), lambda r: (r, 0)),
            pl.BlockSpec((b, d), lambda r: (0, 0)),
            pl.BlockSpec((1, b), lambda r: (0, 0)),
        ],
        out_specs=pl.BlockSpec((bm, b), lambda r: (r, 0)),
        out_shape=jax.ShapeDtypeStruct((b, b), jnp.float32),
        scratch_shapes=[pltpu.VMEM((b, d), jnp.float32)],
    )(rows_i, rows_j, mask_i, mask_j, bias)


def kernel(i_idx, j_idx, wi, wj, bi, bj):
    vocab, d = wi.shape
    b = i_idx.shape[0]
    i32 = i_idx.astype(jnp.int32)
    j32 = j_idx.astype(jnp.int32)
    wi2 = wi.reshape(vocab // 2, 2 * d)
    wj2 = wj.reshape(vocab // 2, 2 * d)
    nbr = (vocab + 127) // 128
    bi128 = jnp.pad(bi.reshape(-1), (0, nbr * 128 - vocab)).reshape(nbr, 128)
    bj128 = jnp.pad(bj.reshape(-1), (0, nbr * 128 - vocab)).reshape(nbr, 128)
    rows_i, rows_j, bias = _sc_gather(i32, j32, wi2, wj2, bi128, bj128)
    mask_i = jnp.broadcast_to((i32 & 1).astype(jnp.float32)[:, None], (b, d))
    mask_j = jnp.broadcast_to((j32 & 1).astype(jnp.float32)[:, None], (b, d))
    return _tc_matmul(rows_i, rows_j, mask_i, mask_j, bias.reshape(1, -1))


# speedup vs baseline: 4.7783x; 4.7783x over previous
import jax
import jax.numpy as jnp
from jax import lax
from jax.experimental import pallas as pl
from jax.experimental.pallas import tpu as pltpu
from jax.experimental.pallas import tpu_sc as plsc

_NC, _NS = 2, 16
_NW = _NC * _NS
_L = 16
_FIRE = 8


def _lane_scalar(vec, k):
    sel = jnp.where(lax.iota(jnp.int32, _L) == k, vec, 0)
    return lax.reduce_max(sel, axes=(0,))


def _lane_splat(vec, k):
    return jnp.take(vec, jnp.zeros((_L,), jnp.int32) + k)


def _extract_column(stage, lane_v, col_v, cols_v, d):
    for m in range(d // _L):
        rid = lax.iota(jnp.int32, _L) + m * _L
        vals = plsc.load_gather(stage, [rid, lane_v])
        plsc.store_scatter(cols_v, [rid, col_v], vals)


def _gather_body(i_idx_hbm, j_idx_hbm, wit_hbm, wjt_hbm, bi128_hbm, bj128_hbm,
                 out_it, out_jt, out_bias,
                 idx_i_v, idx_j_v, brow_i_v, brow_j_v,
                 stage_v, cols_i_v, cols_j_v,
                 browi_v, browj_v, bias_v, sem, bsem):
    bpw = idx_i_v.shape[0]
    d = wit_hbm.shape[0]
    wid = lax.axis_index("s") * _NC + lax.axis_index("c")
    base = wid * bpw
    pltpu.sync_copy(i_idx_hbm.at[pl.ds(base, bpw)], idx_i_v)
    pltpu.sync_copy(j_idx_hbm.at[pl.ds(base, bpw)], idx_j_v)
    for k in range(bpw // _L):
        sl = pl.ds(k * _L, _L)
        brow_i_v[sl] = lax.shift_right_logical(idx_i_v[sl], 7)
        brow_j_v[sl] = lax.shift_right_logical(idx_j_v[sl], 7)
    cb = pltpu.async_copy(bi128_hbm.at[brow_i_v], browi_v, bsem)
    cb2 = pltpu.async_copy(bj128_hbm.at[brow_j_v], browj_v, bsem)

    def _table_loop(tab_hbm, idx_v, cols_v):
        def step(c0, _):
            start = pl.multiple_of(c0 * _L, _L)
            chunk = idx_v[pl.ds(start, _L)]
            qv = lax.shift_right_logical(chunk, 7) * 128
            lanes = chunk & 127
            colbase = c0 * _L
            for w in range(_L // _FIRE):
                copies = []
                for k in range(_FIRE):
                    q = pl.multiple_of(_lane_scalar(qv, w * _FIRE + k), 128)
                    copies.append(pltpu.async_copy(
                        tab_hbm.at[:, pl.ds(q, 128)], stage_v.at[k], sem))
                for k in range(_FIRE):
                    copies[k].wait()
                    lane_v = _lane_splat(lanes, w * _FIRE + k)
                    col_v = (colbase + w * _FIRE + k) + jnp.zeros(
                        (_L,), jnp.int32)
                    _extract_column(stage_v.at[k], lane_v, col_v, cols_v, d)
            return ()

        lax.fori_loop(0, bpw // _L, step, (), unroll=False)

    _table_loop(wit_hbm, idx_i_v, cols_i_v)
    _table_loop(wjt_hbm, idx_j_v, cols_j_v)
    cb.wait()
    cb2.wait()
    for k in range(bpw // _L):
        sl = pl.ds(k * _L, _L)
        rid = lax.iota(jnp.int32, _L) + k * _L
        bi_vals = plsc.load_gather(browi_v, [rid, idx_i_v[sl] & 127])
        bj_vals = plsc.load_gather(browj_v, [rid, idx_j_v[sl] & 127])
        bias_v[sl] = bi_vals + bj_vals
    pltpu.sync_copy(cols_i_v, out_it.at[:, pl.ds(base, bpw)])
    pltpu.sync_copy(cols_j_v, out_jt.at[:, pl.ds(base, bpw)])
    pltpu.sync_copy(bias_v, out_bias.at[pl.ds(base, bpw)])


def _sc_gather(i_idx, j_idx, wit, wjt, bi128, bj128):
    b = i_idx.shape[0]
    d = wit.shape[0]
    bpw = b // _NW
    mesh = plsc.VectorSubcoreMesh(core_axis_name="c", subcore_axis_name="s")
    return pl.kernel(
        _gather_body,
        out_type=(
            jax.ShapeDtypeStruct((d, b), jnp.float32),
            jax.ShapeDtypeStruct((d, b), jnp.float32),
            jax.ShapeDtypeStruct((b,), jnp.float32),
        ),
        mesh=mesh,
        scratch_types=(
            pltpu.VMEM((bpw,), jnp.int32),
            pltpu.VMEM((bpw,), jnp.int32),
            pltpu.VMEM((bpw,), jnp.int32),
            pltpu.VMEM((bpw,), jnp.int32),
            pltpu.VMEM((_FIRE, d, 128), jnp.float32),
            pltpu.VMEM((d, bpw), jnp.float32),
            pltpu.VMEM((d, bpw), jnp.float32),
            pltpu.VMEM((bpw, 128), jnp.float32),
            pltpu.VMEM((bpw, 128), jnp.float32),
            pltpu.VMEM((bpw,), jnp.float32),
            pltpu.SemaphoreType.DMA,
            pltpu.SemaphoreType.DMA,
        ),
        compiler_params=pltpu.CompilerParams(needs_layout_passes=False),
    )(i_idx, j_idx, wit, wjt, bi128, bj128)


def _matmul_body(at_ref, bt_ref, bias_ref, o_ref):
    o_ref[...] = lax.dot_general(
        at_ref[...], bt_ref[...], (((0,), (0,)), ((), ())),
        preferred_element_type=jnp.float32,
    ) + bias_ref[...]


def _tc_matmul(rows_it, rows_jt, bias):
    d, b = rows_it.shape
    bm = 512
    return pl.pallas_call(
        _matmul_body,
        grid=(b // bm,),
        in_specs=[
            pl.BlockSpec((d, bm), lambda r: (0, r)),
            pl.BlockSpec((d, b), lambda r: (0, 0)),
            pl.BlockSpec((1, b), lambda r: (0, 0)),
        ],
        out_specs=pl.BlockSpec((bm, b), lambda r: (r, 0)),
        out_shape=jax.ShapeDtypeStruct((b, b), jnp.float32),
    )(rows_it, rows_jt, bias)


def kernel(i_idx, j_idx, wi, wj, bi, bj):
    vocab, d = wi.shape
    i32 = i_idx.astype(jnp.int32)
    j32 = j_idx.astype(jnp.int32)
    wit = wi.T
    wjt = wj.T
    nbr = (vocab + 127) // 128
    bi128 = jnp.pad(bi.reshape(-1), (0, nbr * 128 - vocab)).reshape(nbr, 128)
    bj128 = jnp.pad(bj.reshape(-1), (0, nbr * 128 - vocab)).reshape(nbr, 128)
    rows_it, rows_jt, bias = _sc_gather(i32, j32, wit, wjt, bi128, bj128)
    return _tc_matmul(rows_it, rows_jt, bias.reshape(1, -1))


# speedup vs baseline: 5.7040x; 1.1937x over previous
import jax
import jax.numpy as jnp
from jax import lax
from jax.experimental import pallas as pl
from jax.experimental.pallas import tpu as pltpu
from jax.experimental.pallas import tpu_sc as plsc

_NC, _NS = 2, 16
_NW = _NC * _NS
_L = 16
_F = 4
_RING = 3
_CHUNKS = 4


def _lane_scalar(vec, k):
    sel = jnp.where(lax.iota(jnp.int32, _L) == k, vec, 0)
    return lax.reduce_max(sel, axes=(0,))


def _lane_splat(vec, k):
    return jnp.take(vec, jnp.zeros((_L,), jnp.int32) + k)


def _extract_column(stage, lane_v, col_v, cols_v, d):
    for m in range(d // _L):
        rid = lax.iota(jnp.int32, _L) + m * _L
        vals = plsc.load_gather(stage, [rid, lane_v])
        plsc.store_scatter(cols_v, [rid, col_v], vals)


def _extract_column_flat(stage, lane_v, col_v, cols_flat, d, bpw):
    for m in range(d // _L):
        rid = lax.iota(jnp.int32, _L) + m * _L
        vals = plsc.load_gather(stage, [rid, lane_v])
        plsc.store_scatter(cols_flat, [rid * bpw + col_v], vals)


def _one_body(idx_hbm, tabt_hbm, out_t, idx_v, stage_v, cols_v, sem):
    bpw = idx_v.shape[0]
    d = tabt_hbm.shape[0]
    flat = len(out_t.shape) == 1
    wid = lax.axis_index("s") * _NC + lax.axis_index("c")
    base = wid * bpw
    pltpu.sync_copy(idx_hbm.at[pl.ds(base, bpw)], idx_v)

    def step(c0, _):
        start = pl.multiple_of(c0 * _L, _L)
        chunk = idx_v[pl.ds(start, _L)]
        qv = lax.shift_right_logical(chunk, 7) * 128
        lanes = chunk & 127
        colbase = c0 * _L

        def fire(w):
            slot0 = (w % _RING) * _F
            cs = []
            for k in range(_F):
                q = pl.multiple_of(_lane_scalar(qv, w * _F + k), 128)
                cs.append(pltpu.async_copy(
                    tabt_hbm.at[:, pl.ds(q, 128)],
                    stage_v.at[slot0 + k], sem))
            return cs

        nw = _L // _F
        copies = [None] * nw
        copies[0] = fire(0)
        copies[1] = fire(1)
        for w in range(nw):
            if w + 2 < nw:
                copies[w + 2] = fire(w + 2)
            slot0 = (w % _RING) * _F
            for k in range(_F):
                copies[w][k].wait()
                lane_v = _lane_splat(lanes, w * _F + k)
                col_v = (colbase + w * _F + k) + jnp.zeros((_L,), jnp.int32)
                if flat:
                    _extract_column_flat(stage_v.at[slot0 + k], lane_v,
                                         col_v, cols_v, d, bpw)
                else:
                    _extract_column(stage_v.at[slot0 + k], lane_v, col_v,
                                    cols_v, d)
        return ()

    lax.fori_loop(0, bpw // _L, step, (), unroll=False)
    if flat:
        pltpu.sync_copy(cols_v, out_t.at[pl.ds(base * d, d * bpw)])
    else:
        pltpu.sync_copy(cols_v, out_t.at[:, pl.ds(base, bpw)])


def _sc_gather_one(idxs, tabt, flat=False):
    n = idxs.shape[0]
    d = tabt.shape[0]
    bpw = n // _NW
    mesh = plsc.VectorSubcoreMesh(core_axis_name="c", subcore_axis_name="s")
    if flat:
        out_type = jax.ShapeDtypeStruct((d * n,), jnp.float32)
        cols = pltpu.VMEM((d * bpw,), jnp.float32)
    else:
        out_type = jax.ShapeDtypeStruct((d, n), jnp.float32)
        cols = pltpu.VMEM((d, bpw), jnp.float32)
    out = pl.kernel(
        _one_body,
        out_type=out_type,
        mesh=mesh,
        scratch_types=(
            pltpu.VMEM((bpw,), jnp.int32),
            pltpu.VMEM((_RING * _F, d, 128), jnp.float32),
            cols,
            pltpu.SemaphoreType.DMA,
        ),
        compiler_params=pltpu.CompilerParams(needs_layout_passes=False),
    )(idxs, tabt)
    if flat:
        out = out.reshape(_NW, d, bpw).transpose(1, 0, 2).reshape(d, n)
    return out


def _bias_body(i_idx_hbm, j_idx_hbm, bi16_hbm, bj16_hbm, out_bias,
               idx_i_v, idx_j_v, brow_i_v, brow_j_v,
               browi_v, browj_v, bias_v, sem):
    bpw = idx_i_v.shape[0]
    wid = lax.axis_index("s") * _NC + lax.axis_index("c")
    base = wid * bpw
    pltpu.sync_copy(i_idx_hbm.at[pl.ds(base, bpw)], idx_i_v)
    pltpu.sync_copy(j_idx_hbm.at[pl.ds(base, bpw)], idx_j_v)
    for k in range(bpw // _L):
        sl = pl.ds(k * _L, _L)
        brow_i_v[sl] = lax.shift_right_logical(idx_i_v[sl], 4)
        brow_j_v[sl] = lax.shift_right_logical(idx_j_v[sl], 4)
    c1 = pltpu.async_copy(bi16_hbm.at[brow_i_v], browi_v, sem)
    c2 = pltpu.async_copy(bj16_hbm.at[brow_j_v], browj_v, sem)
    c1.wait()
    c2.wait()
    for k in range(bpw // _L):
        sl = pl.ds(k * _L, _L)
        rid = lax.iota(jnp.int32, _L) + k * _L
        bi_vals = plsc.load_gather(browi_v, [rid, idx_i_v[sl] & 15])
        bj_vals = plsc.load_gather(browj_v, [rid, idx_j_v[sl] & 15])
        bias_v[sl] = bi_vals + bj_vals
    pltpu.sync_copy(bias_v, out_bias.at[pl.ds(base, bpw)])


def _sc_gather_bias(i_idx, j_idx, bi16, bj16):
    b = i_idx.shape[0]
    bpw = b // _NW
    mesh = plsc.VectorSubcoreMesh(core_axis_name="c", subcore_axis_name="s")
    return pl.kernel(
        _bias_body,
        out_type=jax.ShapeDtypeStruct((b,), jnp.float32),
        mesh=mesh,
        scratch_types=(
            pltpu.VMEM((bpw,), jnp.int32),
            pltpu.VMEM((bpw,), jnp.int32),
            pltpu.VMEM((bpw,), jnp.int32),
            pltpu.VMEM((bpw,), jnp.int32),
            pltpu.VMEM((bpw, _L), jnp.float32),
            pltpu.VMEM((bpw, _L), jnp.float32),
            pltpu.VMEM((bpw,), jnp.float32),
            pltpu.SemaphoreType.DMA,
        ),
        compiler_params=pltpu.CompilerParams(
            use_tc_tiling_on_sc=False, needs_layout_passes=False),
    )(i_idx, j_idx, bi16, bj16)


def _mm_first_body(at_ref, bt_ref, bias_ref, o_ref):
    o_ref[...] = lax.dot_general(
        at_ref[...], bt_ref[...], (((0,), (0,)), ((), ())),
        preferred_element_type=jnp.float32,
    ) + bias_ref[...]


def _mm_chain_body(acc_ref, at_ref, bt_ref, bias_ref, o_ref):
    del acc_ref
    o_ref[...] = lax.dot_general(
        at_ref[...], bt_ref[...], (((0,), (0,)), ((), ())),
        preferred_element_type=jnp.float32,
    ) + bias_ref[...]


def _tc_matmul_chunk(acc, at_k, bt, bias, k):
    d, bm = at_k.shape
    b = bt.shape[1]
    out_shape = jax.ShapeDtypeStruct((b, b), jnp.float32)
    data_specs = [
        pl.BlockSpec((d, bm), lambda r: (0, 0)),
        pl.BlockSpec((d, b), lambda r: (0, 0)),
        pl.BlockSpec((1, b), lambda r: (0, 0)),
    ]
    out_spec = pl.BlockSpec((bm, b), lambda r: (k, 0))
    if acc is None:
        return pl.pallas_call(
            _mm_first_body,
            grid=(1,),
            in_specs=data_specs,
            out_specs=out_spec,
            out_shape=out_shape,
        )(at_k, bt, bias)
    return pl.pallas_call(
        _mm_chain_body,
        grid=(1,),
        in_specs=[pl.BlockSpec(memory_space=pltpu.MemorySpace.HBM)]
        + data_specs,
        out_specs=out_spec,
        out_shape=out_shape,
        input_output_aliases={0: 0},
    )(acc, at_k, bt, bias)


def kernel(i_idx, j_idx, wi, wj, bi, bj):
    vocab, d = wi.shape
    b = i_idx.shape[0]
    i32 = i_idx.astype(jnp.int32)
    j32 = j_idx.astype(jnp.int32)
    bias = _sc_gather_bias(i32, j32, bi.reshape(vocab // _L, _L),
                           bj.reshape(vocab // _L, _L))
    bias_row = bias.reshape(1, -1)
    rows_jt = _sc_gather_one(j32, wj.T)
    bm = b // _CHUNKS
    out = None
    for k in range(_CHUNKS):
        at_k = _sc_gather_one(i32[k * bm:(k + 1) * bm], wi.T, flat=True)
        out = _tc_matmul_chunk(out, at_k, rows_jt, bias_row, k)
    return out


# speedup vs baseline: 7.8652x; 1.3789x over previous
import jax
import jax.numpy as jnp
from jax import lax
from jax.experimental import pallas as pl
from jax.experimental.pallas import tpu as pltpu
from jax.experimental.pallas import tpu_sc as plsc

_NC, _NS = 2, 16
_NW = _NC * _NS
_L = 16
_F = 4
_W = 4
_RING = 3


def _lane_scalar(vec, k):
    sel = jnp.where(lax.iota(jnp.int32, _L) == k, vec, 0)
    return lax.reduce_max(sel, axes=(0,))


def _lane_splat(vec, k):
    return jnp.take(vec, jnp.zeros((_L,), jnp.int32) + k)


def _extract_column(stage, lane_v, col_v, cols_v, d):
    for m in range(d // _L):
        rid = lax.iota(jnp.int32, _L) + m * _L
        vals = plsc.load_gather(stage, [rid, lane_v])
        plsc.store_scatter(cols_v, [rid, col_v], vals)


def _weights_body(i_idx_hbm, j_idx_hbm, wit_hbm, wjt_hbm,
                  out_it, out_jt,
                  idx_i_v, idx_j_v, stage_v, cols_i_v, cols_j_v, sem):
    bpw = idx_i_v.shape[0]
    d = wit_hbm.shape[0]
    wid = lax.axis_index("s") * _NC + lax.axis_index("c")
    base = wid * bpw
    pltpu.sync_copy(i_idx_hbm.at[pl.ds(base, bpw)], idx_i_v)
    pltpu.sync_copy(j_idx_hbm.at[pl.ds(base, bpw)], idx_j_v)

    def step(c0, _):
        start = pl.multiple_of(c0 * _L, _L)
        ci = idx_i_v[pl.ds(start, _L)]
        cj = idx_j_v[pl.ds(start, _L)]
        colbase = c0 * _L
        waves = []
        for w in range(_L // _F):
            waves.append((wit_hbm, lax.shift_right_logical(ci, 7) * 128,
                          ci & 127, cols_i_v, w * _F))
            waves.append((wjt_hbm, lax.shift_right_logical(cj, 7) * 128,
                          cj & 127, cols_j_v, w * _F))

        def fire(g):
            tab_hbm, qv, lanes, cols_v, kb = waves[g]
            slot0 = (g % _RING) * _F
            cs = []
            for k in range(_F):
                q = pl.multiple_of(_lane_scalar(qv, kb + k), 128)
                cs.append(pltpu.async_copy(
                    tab_hbm.at[:, pl.ds(q, 128)],
                    stage_v.at[slot0 + k], sem))
            return cs

        ng = len(waves)
        copies = [None] * ng
        copies[0] = fire(0)
        copies[1] = fire(1)
        for g in range(ng):
            if g + 2 < ng:
                copies[g + 2] = fire(g + 2)
            _, _, lanes, cols_v, kb = waves[g]
            slot0 = (g % _RING) * _F
            for k in range(_F):
                copies[g][k].wait()
                lane_v = _lane_splat(lanes, kb + k)
                col_v = (colbase + kb + k) + jnp.zeros((_L,), jnp.int32)
                _extract_column(stage_v.at[slot0 + k], lane_v, col_v,
                                cols_v, d)
        return ()

    lax.fori_loop(0, bpw // _L, step, (), unroll=False)
    pltpu.sync_copy(cols_i_v, out_it.at[:, pl.ds(base, bpw)])
    pltpu.sync_copy(cols_j_v, out_jt.at[:, pl.ds(base, bpw)])


def _sc_gather_weights(i_idx, j_idx, wit, wjt):
    b = i_idx.shape[0]
    d = wit.shape[0]
    bpw = b // _NW
    mesh = plsc.VectorSubcoreMesh(core_axis_name="c", subcore_axis_name="s")
    return pl.kernel(
        _weights_body,
        out_type=(
            jax.ShapeDtypeStruct((d, b), jnp.float32),
            jax.ShapeDtypeStruct((d, b), jnp.float32),
        ),
        mesh=mesh,
        scratch_types=(
            pltpu.VMEM((bpw,), jnp.int32),
            pltpu.VMEM((bpw,), jnp.int32),
            pltpu.VMEM((_RING * _F, d, 128), jnp.float32),
            pltpu.VMEM((d, bpw), jnp.float32),
            pltpu.VMEM((d, bpw), jnp.float32),
            pltpu.SemaphoreType.DMA,
        ),
        compiler_params=pltpu.CompilerParams(needs_layout_passes=False),
    )(i_idx, j_idx, wit, wjt)


def _bias_body(i_idx_hbm, j_idx_hbm, bi16_hbm, bj16_hbm, out_bias,
               idx_i_v, idx_j_v, brow_i_v, brow_j_v,
               browi_v, browj_v, bias_v, sem):
    bpw = idx_i_v.shape[0]
    wid = lax.axis_index("s") * _NC + lax.axis_index("c")
    base = wid * bpw
    pltpu.sync_copy(i_idx_hbm.at[pl.ds(base, bpw)], idx_i_v)
    pltpu.sync_copy(j_idx_hbm.at[pl.ds(base, bpw)], idx_j_v)
    for k in range(bpw // _L):
        sl = pl.ds(k * _L, _L)
        brow_i_v[sl] = lax.shift_right_logical(idx_i_v[sl], 4)
        brow_j_v[sl] = lax.shift_right_logical(idx_j_v[sl], 4)
    c1 = pltpu.async_copy(bi16_hbm.at[brow_i_v], browi_v, sem)
    c2 = pltpu.async_copy(bj16_hbm.at[brow_j_v], browj_v, sem)
    c1.wait()
    c2.wait()
    for k in range(bpw // _L):
        sl = pl.ds(k * _L, _L)
        rid = lax.iota(jnp.int32, _L) + k * _L
        bi_vals = plsc.load_gather(browi_v, [rid, idx_i_v[sl] & 15])
        bj_vals = plsc.load_gather(browj_v, [rid, idx_j_v[sl] & 15])
        bias_v[sl] = bi_vals + bj_vals
    pltpu.sync_copy(bias_v, out_bias.at[pl.ds(base, bpw)])


def _sc_gather_bias(i_idx, j_idx, bi16, bj16):
    b = i_idx.shape[0]
    bpw = b // _NW
    mesh = plsc.VectorSubcoreMesh(core_axis_name="c", subcore_axis_name="s")
    return pl.kernel(
        _bias_body,
        out_type=jax.ShapeDtypeStruct((b,), jnp.float32),
        mesh=mesh,
        scratch_types=(
            pltpu.VMEM((bpw,), jnp.int32),
            pltpu.VMEM((bpw,), jnp.int32),
            pltpu.VMEM((bpw,), jnp.int32),
            pltpu.VMEM((bpw,), jnp.int32),
            pltpu.VMEM((bpw, _L), jnp.float32),
            pltpu.VMEM((bpw, _L), jnp.float32),
            pltpu.VMEM((bpw,), jnp.float32),
            pltpu.SemaphoreType.DMA,
        ),
        compiler_params=pltpu.CompilerParams(
            use_tc_tiling_on_sc=False, needs_layout_passes=False),
    )(i_idx, j_idx, bi16, bj16)


def _matmul_body(at_ref, bt_ref, bias_ref, o_ref):
    o_ref[...] = lax.dot_general(
        at_ref[...], bt_ref[...], (((0,), (0,)), ((), ())),
        preferred_element_type=jnp.float32,
    ) + bias_ref[...]


def _tc_matmul(rows_it, rows_jt, bias):
    d, b = rows_it.shape
    bm = 512
    return pl.pallas_call(
        _matmul_body,
        grid=(b // bm,),
        in_specs=[
            pl.BlockSpec((d, bm), lambda r: (0, r)),
            pl.BlockSpec((d, b), lambda r: (0, 0)),
            pl.BlockSpec((1, b), lambda r: (0, 0)),
        ],
        out_specs=pl.BlockSpec((bm, b), lambda r: (r, 0)),
        out_shape=jax.ShapeDtypeStruct((b, b), jnp.float32),
    )(rows_it, rows_jt, bias)


def kernel(i_idx, j_idx, wi, wj, bi, bj):
    vocab, d = wi.shape
    i32 = i_idx.astype(jnp.int32)
    j32 = j_idx.astype(jnp.int32)
    rows_it, rows_jt = _sc_gather_weights(i32, j32, wi.T, wj.T)
    bias = _sc_gather_bias(i32, j32, bi.reshape(vocab // _L, _L),
                           bj.reshape(vocab // _L, _L))
    return _tc_matmul(rows_it, rows_jt, bias.reshape(1, -1))
